# Initial kernel scaffold; baseline (speedup 1.0000x reference)
import jax
import jax.numpy as jnp
from jax.experimental import pallas as pl


def kernel(x, proposalN):
    raise NotImplementedError("write your pallas kernel here")


# speedup vs baseline: 5.5683x; 5.5683x over previous
import jax
import jax.numpy as jnp
import numpy as np
from jax.experimental import pallas as pl
from jax.experimental.pallas import tpu as pltpu

_RATIOS = [[4, 4], [3, 5], [5, 3], [6, 6], [5, 7], [7, 5], [8, 8], [6, 10],
           [10, 6], [7, 9], [9, 7], [7, 10], [10, 7]]
_STRIDE = 16
_FM = 28
_CAT_NUMS = [2, 3, 2]
_IOU_THRESH = 0.25
_GROUPS = [(0, 1873), (1873, 3458), (3458, 6489)]
_NWIN = 6489
_GP = 3072
_GR, _GC = 24, 128
_CB = 6
_CBS = 128
_BIG_I = np.int32(1 << 30)


def _window_coords():
    out = []
    for (kh, kw) in _RATIOS:
        for i in range(_FM - kh + 1):
            for j in range(_FM - kw + 1):
                out.append([j * _STRIDE, i * _STRIDE,
                            (j + kw) * _STRIDE - 1, (i + kh) * _STRIDE - 1])
    return np.array(out, dtype=np.float64)


_COORDS = _window_coords()
_AREAS = ((_COORDS[:, 2] - _COORDS[:, 0] + 1.0)
          * (_COORDS[:, 3] - _COORDS[:, 1] + 1.0))


def _padded_group(vec, pad_val, dtype):
    rows = []
    for (lo, hi) in _GROUPS:
        seg = np.full((_GP,), pad_val, dtype=dtype)
        seg[: hi - lo] = vec[lo:hi].astype(dtype)
        rows.append(seg)
    return np.stack(rows, axis=0).reshape(3, _GR, _GC)


_IOTA_P = _padded_group(np.arange(_NWIN), _BIG_I, np.int32)
_X0_P = _padded_group(_COORDS[:, 0], 1e9, np.float32)
_Y0_P = _padded_group(_COORDS[:, 1], 1e9, np.float32)
_X1_P = _padded_group(_COORDS[:, 2], -1e9, np.float32)
_Y1_P = _padded_group(_COORDS[:, 3], -1e9, np.float32)
_AR_P = _padded_group(_AREAS, 1.0, np.float32)


def _scores_body(x_ref, *refs):
    out_refs, acc_ref = refs[:-1], refs[-1]
    c = pl.program_id(1)

    @pl.when(c == 0)
    def _init():
        acc_ref[...] = jnp.zeros((_FM, _FM), jnp.float32)

    acc_ref[...] = acc_ref[...] + jnp.sum(x_ref[0], axis=0)

    @pl.when(c == _CB - 1)
    def _finish():
        s = acc_ref[...]
        hs = {1: s}
        cur = s
        for k in range(2, 11):
            cur = cur[:, : _FM + 1 - k] + s[:, k - 1:]
            hs[k] = cur
        for r, (kh, kw) in enumerate(_RATIOS):
            h = hs[kw]
            v = h
            for k in range(2, kh + 1):
                v = v[: _FM + 1 - k, :] + h[k - 1:, :]
            out_refs[r][0] = v / float(kh * kw)


def _nms_body(sp_ref, iota_ref, x0_ref, y0_ref, x1_ref, y1_ref, ar_ref,
              idx_ref, sc_ref):
    orig = sp_ref[0, 0]
    iota = iota_ref[0]
    x0 = x0_ref[0]
    y0 = y0_ref[0]
    x1 = x1_ref[0]
    y1 = y1_ref[0]
    ar = ar_ref[0]
    lane = jax.lax.broadcasted_iota(jnp.int32, (1, _GC), 1)

    ms = orig
    last = jnp.min(iota)
    idx_acc = jnp.zeros((1, _GC), jnp.int32)
    sc_acc = jnp.zeros((1, _GC), jnp.float32)
    neg = jnp.float32(-jnp.inf)
    for t in range(max(_CAT_NUMS)):
        m = jnp.max(ms)
        valid = m != neg
        pick = jnp.min(jnp.where(ms == m, iota, _BIG_I))
        idx = jnp.where(valid, pick, last)
        eqi = iota == idx
        bx0 = jnp.sum(jnp.where(eqi, x0, 0.0))
        by0 = jnp.sum(jnp.where(eqi, y0, 0.0))
        bx1 = jnp.sum(jnp.where(eqi, x1, 0.0))
        by1 = jnp.sum(jnp.where(eqi, y1, 0.0))
        bar = jnp.sum(jnp.where(eqi, ar, 0.0))
        sel = jnp.sum(jnp.where(eqi, orig, 0.0))
        idx_acc = jnp.where(lane == t, idx, idx_acc)
        sc_acc = jnp.where(lane == t, sel, sc_acc)
        lx = jnp.minimum(x1, bx1) - jnp.maximum(x0, bx0) + 1.0
        ly = jnp.minimum(y1, by1) - jnp.maximum(y0, by0) + 1.0
        inter = jnp.where((lx < 0) | (ly < 0), 0.0, lx * ly)
        iou = inter / (ar + bar - inter)
        kill = (iou > _IOU_THRESH) | eqi
        ms = jnp.where(jnp.logical_and(valid, kill), neg, ms)
        last = idx
    idx_ref[...] = idx_acc.reshape(1, 1, 1, _GC)
    sc_ref[...] = sc_acc.reshape(1, 1, 1, _GC)


def _run(x, proposalN):
    b = x.shape[0]
    pooled = pl.pallas_call(
        _scores_body,
        grid=(b, _CB),
        in_specs=[pl.BlockSpec((1, _CBS, _FM, _FM), lambda i, c: (i, c, 0, 0))],
        out_specs=[pl.BlockSpec((1, _FM + 1 - kh, _FM + 1 - kw),
                                lambda i, c: (i, 0, 0))
                   for (kh, kw) in _RATIOS],
        out_shape=[jax.ShapeDtypeStruct((b, _FM + 1 - kh, _FM + 1 - kw),
                                        jnp.float32)
                   for (kh, kw) in _RATIOS],
        scratch_shapes=[pltpu.VMEM((_FM, _FM), jnp.float32)],
    )(x)
    ws = jnp.concatenate([o.reshape(b, -1) for o in pooled], axis=1)

    segs = []
    for (lo, hi) in _GROUPS:
        segs.append(jnp.pad(ws[:, lo:hi], ((0, 0), (0, _GP - (hi - lo))),
                            constant_values=-np.inf))
    sp = jnp.stack(segs, axis=1).reshape(b, 3, _GR, _GC)

    idx_o, sc_o = pl.pallas_call(
        _nms_body,
        grid=(b, 3),
        in_specs=[
            pl.BlockSpec((1, 1, _GR, _GC), lambda i, g: (i, g, 0, 0)),
            pl.BlockSpec((1, _GR, _GC), lambda i, g: (g, 0, 0)),
            pl.BlockSpec((1, _GR, _GC), lambda i, g: (g, 0, 0)),
            pl.BlockSpec((1, _GR, _GC), lambda i, g: (g, 0, 0)),
            pl.BlockSpec((1, _GR, _GC), lambda i, g: (g, 0, 0)),
            pl.BlockSpec((1, _GR, _GC), lambda i, g: (g, 0, 0)),
            pl.BlockSpec((1, _GR, _GC), lambda i, g: (g, 0, 0)),
        ],
        out_specs=[pl.BlockSpec((1, 1, 1, _GC), lambda i, g: (i, g, 0, 0)),
                   pl.BlockSpec((1, 1, 1, _GC), lambda i, g: (i, g, 0, 0))],
        out_shape=[jax.ShapeDtypeStruct((b, 3, 1, _GC), jnp.int32),
                   jax.ShapeDtypeStruct((b, 3, 1, _GC), jnp.float32)],
    )(sp, jnp.asarray(_IOTA_P), jnp.asarray(_X0_P), jnp.asarray(_Y0_P),
      jnp.asarray(_X1_P), jnp.asarray(_Y1_P), jnp.asarray(_AR_P))

    idxs = idx_o[:, :, 0, :]
    scs = sc_o[:, :, 0, :]
    inds = jnp.concatenate([idxs[:, g, : _CAT_NUMS[g]] for g in range(3)],
                           axis=1)
    ssc = jnp.concatenate([scs[:, g, : _CAT_NUMS[g]] for g in range(3)],
                          axis=1)
    inds = inds + (jnp.asarray(proposalN, jnp.int32) - sum(_CAT_NUMS))
    return inds.astype(jnp.int32), ssc, ws


def kernel(x, proposalN):
    return _run(x, proposalN)
